# Initial kernel scaffold; baseline (speedup 1.0000x reference)
import jax
import jax.numpy as jnp
from jax.experimental import pallas as pl


def kernel(atomic_num, formal_charge, degree, chiral_tag, total_numHs, is_aromatic, hybridization, W_atomic_num, W_formal_charge, W_degree, W_chiral_tag, W_total_numHs, W_is_aromatic, W_hybridization):
    raise NotImplementedError("write your pallas kernel here")


# speedup vs baseline: 8.0017x; 8.0017x over previous
import functools

import jax
import jax.numpy as jnp
from jax import lax
from jax.experimental import pallas as pl
from jax.experimental.pallas import tpu as pltpu
from jax.experimental.pallas import tpu_sc as plsc

N = 100000
D = 32
SIZES = (120, 17, 13, 5, 10, 3, 7)
NT = len(SIZES)

NC = 2
NS = 16
NW = NC * NS
LANES = 16

NPAD = 102400
PER_TILE = NPAD // NW
CHUNK = 400
NCHUNKS = PER_TILE // CHUNK


def _sc_body(*refs):
    idx_hbm = refs[0:NT]
    w_hbm = refs[NT:2 * NT]
    out_hbm = refs[2 * NT]
    idx_v = refs[2 * NT + 1:3 * NT + 1]
    w_v = refs[3 * NT + 1:4 * NT + 1]
    out_v = refs[4 * NT + 1]

    c = lax.axis_index("c")
    s = lax.axis_index("s")
    wid = s * NC + c
    base = wid * PER_TILE

    for t in range(NT):
        pltpu.sync_copy(w_hbm[t], w_v[t])

    iota = lax.iota(jnp.int32, LANES)

    def chunk_body(ci, carry):
        row0 = base + ci * CHUNK
        for t in range(NT):
            pltpu.sync_copy(idx_hbm[t].at[pl.ds(row0, CHUNK)], idx_v[t])

        def group_body(g, carry2):
            fvecs = [idx_v[t][pl.ds(g * LANES, LANES)] * D for t in range(NT)]
            for j in range(LANES):
                acc_lo = jnp.zeros((LANES,), jnp.float32)
                acc_hi = jnp.zeros((LANES,), jnp.float32)
                for t in range(NT):
                    lo = jnp.full((LANES,), fvecs[t][j], jnp.int32) + iota
                    acc_lo = acc_lo + plsc.load_gather(w_v[t], [lo])
                    acc_hi = acc_hi + plsc.load_gather(w_v[t], [lo + LANES])
                a = (g * LANES + j) * D
                out_v[pl.ds(a, LANES)] = acc_lo
                out_v[pl.ds(a + LANES, LANES)] = acc_hi
            return carry2

        lax.fori_loop(0, CHUNK // LANES, group_body, 0)
        pltpu.sync_copy(out_v, out_hbm.at[pl.ds(row0 * D, CHUNK * D)])
        return carry

    lax.fori_loop(0, NCHUNKS, chunk_body, 0)


@jax.jit
def _run(idxs, tables_flat):
    mesh = plsc.VectorSubcoreMesh(
        core_axis_name="c", subcore_axis_name="s",
        num_cores=NC, num_subcores=NS)
    scratch = (
        [pltpu.VMEM((CHUNK,), jnp.int32) for _ in range(NT)]
        + [pltpu.VMEM((SIZES[t] * D,), jnp.float32) for t in range(NT)]
        + [pltpu.VMEM((CHUNK * D,), jnp.float32)]
    )
    fn = pl.kernel(
        _sc_body,
        out_type=jax.ShapeDtypeStruct((NPAD * D,), jnp.float32),
        mesh=mesh,
        scratch_types=scratch,
        compiler_params=pltpu.CompilerParams(needs_layout_passes=False),
    )
    return fn(*idxs, *tables_flat)


def kernel(atomic_num, formal_charge, degree, chiral_tag, total_numHs,
           is_aromatic, hybridization, W_atomic_num, W_formal_charge,
           W_degree, W_chiral_tag, W_total_numHs, W_is_aromatic,
           W_hybridization):
    idxs = [atomic_num, formal_charge, degree, chiral_tag, total_numHs,
            is_aromatic, hybridization]
    tables = [W_atomic_num, W_formal_charge, W_degree, W_chiral_tag,
              W_total_numHs, W_is_aromatic, W_hybridization]
    pad = NPAD - N
    idxs = [jnp.concatenate([i, jnp.zeros((pad,), jnp.int32)]) for i in idxs]
    tables_flat = [w.reshape(-1) for w in tables]
    out = _run(idxs, tables_flat)
    return out.reshape(NPAD, D)[:N]
